# Initial kernel scaffold; baseline (speedup 1.0000x reference)
import jax
import jax.numpy as jnp
from jax.experimental import pallas as pl


def kernel(user_indices, item_indices, edge_index, user_table, item_table, W1, att_src1, att_dst1, b1, W2, att_src2, att_dst2, b2):
    raise NotImplementedError("write your pallas kernel here")


# speedup vs baseline: 10.9590x; 10.9590x over previous
import functools

import jax
import jax.numpy as jnp
from jax import lax
from jax.experimental import pallas as pl
from jax.experimental.pallas import tpu as pltpu
from jax.experimental.pallas import tpu_sc as plsc

NUSERS = 50000
N = 100000
D = 64
H = 4
C = 16
E = 100000
EPAD = 102400
BATCH = 16384
NC = 2
NS = 16
NPAD = 102400
RPT = NPAD // NS
ZRB = 40
ZRD = 1280
CH = 128
f32 = jnp.float32
i32 = jnp.int32

_mesh = plsc.VectorSubcoreMesh(core_axis_name="c", subcore_axis_name="s")


def _zero_shared(acc, zbuf, sid):
  z = jnp.zeros((16,), f32)
  for i in range(ZRB):
    zbuf[i, :] = z

  def zb(k, carry):
    pltpu.sync_copy(zbuf, acc.at[pl.ds(sid * RPT + k * ZRB, ZRB)])
    return carry

  lax.fori_loop(0, RPT // ZRB, zb, 0)


def _edge_alpha_body(s_hbm, t_hbm, src_hbm, dst_hbm, msk_hbm,
                     ee_hbm, asum_hbm,
                     acc, zbuf, idxs, idxd, mbuf, srow, trow, eebuf):
  cid = lax.axis_index("c")
  sid = lax.axis_index("s")
  w = cid * NS + sid
  _zero_shared(acc, zbuf, sid)
  plsc.subcore_barrier()
  per_w = EPAD // (NC * NS)

  def chunk(ci, carry):
    base = w * per_w + ci * CH
    pltpu.sync_copy(src_hbm.at[pl.ds(base, CH)], idxs)
    pltpu.sync_copy(dst_hbm.at[pl.ds(base, CH)], idxd)
    pltpu.sync_copy(msk_hbm.at[pl.ds(base, CH)], mbuf)
    pltpu.sync_copy(s_hbm.at[idxs], srow)
    pltpu.sync_copy(t_hbm.at[idxd], trow)
    for j in range(CH // 16):
      mrow = mbuf[pl.ds(j * 16, 16)]
      for l in range(16):
        i = j * 16 + l
        a = srow[i, :] + trow[i, :]
        a = jnp.maximum(a, 0.2 * a)
        eebuf[i, :] = jnp.exp(a) * mrow[l]
    pltpu.sync_copy(eebuf, ee_hbm.at[pl.ds(base, CH)])
    pltpu.sync_copy(eebuf, acc.at[idxd], add=True)
    return carry

  lax.fori_loop(0, per_w // CH, chunk, 0)
  plsc.subcore_barrier()

  def dump(k, carry):
    off = sid * RPT + k * ZRD
    pltpu.sync_copy(acc.at[pl.ds(off, ZRD)],
                    asum_hbm.at[pl.ds(cid * NPAD + off, ZRD)])
    return carry

  lax.fori_loop(0, RPT // ZRD, dump, 0)


_edge_alpha = functools.partial(
    pl.kernel,
    out_type=(jax.ShapeDtypeStruct((EPAD, 16), f32),
              jax.ShapeDtypeStruct((NC * NPAD, 16), f32)),
    mesh=_mesh,
    scratch_types=[
        pltpu.VMEM_SHARED((NPAD, 16), f32),
        pltpu.VMEM((ZRB, 16), f32),
        pltpu.VMEM((CH,), i32),
        pltpu.VMEM((CH,), i32),
        pltpu.VMEM((CH,), f32),
        pltpu.VMEM((CH, 16), f32),
        pltpu.VMEM((CH, 16), f32),
        pltpu.VMEM((CH, 16), f32),
    ],
    compiler_params=pltpu.CompilerParams(use_tc_tiling_on_sc=False),
)(_edge_alpha_body)


def _msg_body(hh_hbm, ee_hbm, src_hbm, dst_hbm, out_hbm,
              acc, zbuf, idxs, idxd, hrow, eebuf, msgbuf):
  cid = lax.axis_index("c")
  sid = lax.axis_index("s")
  w = cid * NS + sid
  per_w = EPAD // (NC * NS)

  for head in range(H):
    rowoff = head * N
    outoff = (cid * H + head) * NPAD
    _zero_shared(acc, zbuf, sid)
    plsc.subcore_barrier()

    def chunk(ci, carry):
      base = w * per_w + ci * CH
      pltpu.sync_copy(src_hbm.at[pl.ds(base, CH)], idxs)
      pltpu.sync_copy(dst_hbm.at[pl.ds(base, CH)], idxd)
      for j in range(CH // 16):
        idxs[pl.ds(j * 16, 16)] = idxs[pl.ds(j * 16, 16)] + rowoff
      pltpu.sync_copy(hh_hbm.at[idxs], hrow)
      pltpu.sync_copy(ee_hbm.at[pl.ds(base, CH)], eebuf)
      for i in range(CH):
        es = eebuf[i, :][head]
        msgbuf[i, :] = hrow[i, :] * es
      pltpu.sync_copy(msgbuf, acc.at[idxd], add=True)
      return carry

    lax.fori_loop(0, per_w // CH, chunk, 0)
    plsc.subcore_barrier()

    def dump(k, carry):
      off = sid * RPT + k * ZRD
      pltpu.sync_copy(acc.at[pl.ds(off, ZRD)],
                      out_hbm.at[pl.ds(outoff + off, ZRD)])
      return carry

    lax.fori_loop(0, RPT // ZRD, dump, 0)
    plsc.subcore_barrier()


_msg = functools.partial(
    pl.kernel,
    out_type=jax.ShapeDtypeStruct((NC * H * NPAD, 16), f32),
    mesh=_mesh,
    scratch_types=[
        pltpu.VMEM_SHARED((NPAD, 16), f32),
        pltpu.VMEM((ZRB, 16), f32),
        pltpu.VMEM((CH,), i32),
        pltpu.VMEM((CH,), i32),
        pltpu.VMEM((CH, 16), f32),
        pltpu.VMEM((CH, 16), f32),
        pltpu.VMEM((CH, 16), f32),
    ],
    compiler_params=pltpu.CompilerParams(use_tc_tiling_on_sc=False),
)(_msg_body)


def _predict_body(x_hbm, ui_hbm, ii_hbm, uout_hbm, vout_hbm,
                  idxu, idxi, urow, vrow):
  cid = lax.axis_index("c")
  sid = lax.axis_index("s")
  w = cid * NS + sid
  per_w = BATCH // (NC * NS)

  def chunk(ci, carry):
    base = w * per_w + ci * CH
    pltpu.sync_copy(ui_hbm.at[pl.ds(base, CH)], idxu)
    pltpu.sync_copy(ii_hbm.at[pl.ds(base, CH)], idxi)
    for j in range(CH // 16):
      idxi[pl.ds(j * 16, 16)] = idxi[pl.ds(j * 16, 16)] + NUSERS
    pltpu.sync_copy(x_hbm.at[idxu], urow)
    pltpu.sync_copy(x_hbm.at[idxi], vrow)
    pltpu.sync_copy(urow, uout_hbm.at[pl.ds(base, CH)])
    pltpu.sync_copy(vrow, vout_hbm.at[pl.ds(base, CH)])
    return carry

  lax.fori_loop(0, per_w // CH, chunk, 0)


_predict = functools.partial(
    pl.kernel,
    out_type=(jax.ShapeDtypeStruct((BATCH, D), f32),
              jax.ShapeDtypeStruct((BATCH, D), f32)),
    mesh=_mesh,
    scratch_types=[
        pltpu.VMEM((CH,), i32),
        pltpu.VMEM((CH,), i32),
        pltpu.VMEM((CH, D), f32),
        pltpu.VMEM((CH, D), f32),
    ],
    compiler_params=pltpu.CompilerParams(use_tc_tiling_on_sc=False),
)(_predict_body)


def _dot_body(u_ref, v_ref, o_ref):
  d = jnp.sum(u_ref[...] * v_ref[...], axis=1, keepdims=True)
  o_ref[...] = 1.0 / (1.0 + jnp.exp(-d))


_dot = pl.pallas_call(
    _dot_body,
    grid=(BATCH // 2048,),
    in_specs=[
        pl.BlockSpec((2048, D), lambda i: (i, 0)),
        pl.BlockSpec((2048, D), lambda i: (i, 0)),
    ],
    out_specs=pl.BlockSpec((2048, 1), lambda i: (i, 0)),
    out_shape=jax.ShapeDtypeStruct((BATCH, 1), f32),
)


R1 = 800


def _seg_matrix():
  return (lax.broadcasted_iota(i32, (D, H), 0) // C
          == lax.broadcasted_iota(i32, (D, H), 1)).astype(f32)


def _elu(x):
  return jnp.where(x > 0, x, jnp.exp(jnp.minimum(x, 0.0)) - 1.0)


def _dense1_body(x_ref, w_ref, asrc_ref, adst_ref,
                 h_ref, s_ref, t_ref, se_ref):
  x = x_ref[...]
  h = jnp.dot(x, w_ref[...], preferred_element_type=f32)
  seg = _seg_matrix()
  a_s = jnp.dot(h * asrc_ref[...], seg, preferred_element_type=f32)
  a_d = jnp.dot(h * adst_ref[...], seg, preferred_element_type=f32)
  sa = a_s + a_d
  sa = jnp.where(sa > 0, sa, 0.2 * sa)
  se_ref[...] = jnp.exp(sa)
  h_ref[...] = h
  z12 = jnp.zeros((x.shape[0], 12), f32)
  s_ref[...] = jnp.concatenate([a_s, z12], axis=1)
  t_ref[...] = jnp.concatenate([a_d, z12], axis=1)


_dense1 = pl.pallas_call(
    _dense1_body,
    grid=(N // R1,),
    in_specs=[
        pl.BlockSpec((R1, D), lambda i: (i, 0)),
        pl.BlockSpec((D, D), lambda i: (0, 0)),
        pl.BlockSpec((1, D), lambda i: (0, 0)),
        pl.BlockSpec((1, D), lambda i: (0, 0)),
    ],
    out_specs=[
        pl.BlockSpec((R1, D), lambda i: (i, 0)),
        pl.BlockSpec((R1, 16), lambda i: (i, 0)),
        pl.BlockSpec((R1, 16), lambda i: (i, 0)),
        pl.BlockSpec((R1, H), lambda i: (i, 0)),
    ],
    out_shape=[
        jax.ShapeDtypeStruct((N, D), f32),
        jax.ShapeDtypeStruct((N, 16), f32),
        jax.ShapeDtypeStruct((N, 16), f32),
        jax.ShapeDtypeStruct((N, H), f32),
    ],
)


def _dense2_body(acca_ref, accb_ref, h1_ref, se_ref, asum0_ref, asum1_ref,
                 b1_ref, w2_ref, as2_ref, ad2_ref,
                 h2_ref, s2_ref, t2_ref, se2_ref):
  se = se_ref[...]
  asum4 = asum0_ref[:, 0:4] + asum1_ref[:, 0:4] + se
  segt = _seg_matrix().T
  acc = acca_ref[...] + accb_ref[...]
  num = acc + jnp.dot(se, segt, preferred_element_type=f32) * h1_ref[...]
  den = jnp.dot(asum4, segt, preferred_element_type=f32) + 1e-16
  x2 = _elu(num / den + b1_ref[...])
  h2 = jnp.dot(x2, w2_ref[...], preferred_element_type=f32)
  a_s = jnp.sum(h2 * as2_ref[...], axis=1, keepdims=True)
  a_d = jnp.sum(h2 * ad2_ref[...], axis=1, keepdims=True)
  sa = a_s + a_d
  sa = jnp.where(sa > 0, sa, 0.2 * sa)
  rr = h2.shape[0]
  h2_ref[...] = h2
  z12 = jnp.zeros((rr, 12), f32)
  s2_ref[...] = jnp.concatenate([jnp.broadcast_to(a_s, (rr, 4)), z12], axis=1)
  t2_ref[...] = jnp.concatenate([jnp.broadcast_to(a_d, (rr, 4)), z12], axis=1)
  se2_ref[...] = jnp.broadcast_to(jnp.exp(sa), (rr, 4))


_dense2 = pl.pallas_call(
    _dense2_body,
    grid=(N // R1,),
    in_specs=[
        pl.BlockSpec((R1, D), lambda i: (i, 0)),
        pl.BlockSpec((R1, D), lambda i: (i, 0)),
        pl.BlockSpec((R1, D), lambda i: (i, 0)),
        pl.BlockSpec((R1, H), lambda i: (i, 0)),
        pl.BlockSpec((R1, 16), lambda i: (i, 0)),
        pl.BlockSpec((R1, 16), lambda i: (i + NPAD // R1, 0)),
        pl.BlockSpec((1, D), lambda i: (0, 0)),
        pl.BlockSpec((D, D), lambda i: (0, 0)),
        pl.BlockSpec((1, D), lambda i: (0, 0)),
        pl.BlockSpec((1, D), lambda i: (0, 0)),
    ],
    out_specs=[
        pl.BlockSpec((R1, D), lambda i: (i, 0)),
        pl.BlockSpec((R1, 16), lambda i: (i, 0)),
        pl.BlockSpec((R1, 16), lambda i: (i, 0)),
        pl.BlockSpec((R1, H), lambda i: (i, 0)),
    ],
    out_shape=[
        jax.ShapeDtypeStruct((N, D), f32),
        jax.ShapeDtypeStruct((N, 16), f32),
        jax.ShapeDtypeStruct((N, 16), f32),
        jax.ShapeDtypeStruct((N, H), f32),
    ],
)


def _dense3_body(acca_ref, accb_ref, asum0_ref, asum1_ref, se2_ref, h2_ref,
                 b2_ref, x3_ref):
  se2 = se2_ref[:, 0:1]
  den = asum0_ref[:, 0:1] + asum1_ref[:, 0:1] + se2 + 1e-16
  num = acca_ref[...] + accb_ref[...] + se2 * h2_ref[...]
  x3_ref[...] = _elu(num / den + b2_ref[...])


_dense3 = pl.pallas_call(
    _dense3_body,
    grid=(N // R1,),
    in_specs=[
        pl.BlockSpec((R1, D), lambda i: (i, 0)),
        pl.BlockSpec((R1, D), lambda i: (i, 0)),
        pl.BlockSpec((R1, 16), lambda i: (i, 0)),
        pl.BlockSpec((R1, 16), lambda i: (i + NPAD // R1, 0)),
        pl.BlockSpec((R1, H), lambda i: (i, 0)),
        pl.BlockSpec((R1, D), lambda i: (i, 0)),
        pl.BlockSpec((1, D), lambda i: (0, 0)),
    ],
    out_specs=pl.BlockSpec((R1, D), lambda i: (i, 0)),
    out_shape=jax.ShapeDtypeStruct((N, D), f32),
)


@jax.jit
def kernel(user_indices, item_indices, edge_index, user_table, item_table,
           W1, att_src1, att_dst1, b1, W2, att_src2, att_dst2, b2):
  x = jnp.concatenate([user_table, item_table], axis=0)
  src = jnp.concatenate([edge_index[0].astype(i32),
                         jnp.zeros((EPAD - E,), i32)])
  dst = jnp.concatenate([edge_index[1].astype(i32),
                         jnp.zeros((EPAD - E,), i32)])
  msk = jnp.concatenate([jnp.ones((E,), f32), jnp.zeros((EPAD - E,), f32)])

  h1, S1, T1, SE1 = _dense1(x, W1, att_src1.reshape(1, D),
                            att_dst1.reshape(1, D))
  ee1, asum1 = _edge_alpha(S1, T1, src, dst, msk)
  hh1 = h1.reshape(N, H, C).transpose(1, 0, 2).reshape(H * N, C)
  acc1f = _msg(hh1, ee1, src, dst).reshape(NC, H, NPAD, C)
  acc1a = acc1f[0, :, :N].transpose(1, 0, 2).reshape(N, D)
  acc1b = acc1f[1, :, :N].transpose(1, 0, 2).reshape(N, D)

  h2, S2, T2, SE2 = _dense2(acc1a, acc1b, h1, SE1, asum1, asum1,
                            b1.reshape(1, D), W2,
                            att_src2.reshape(1, D), att_dst2.reshape(1, D))
  ee2, asum2 = _edge_alpha(S2, T2, src, dst, msk)
  hh2 = h2.reshape(N, H, C).transpose(1, 0, 2).reshape(H * N, C)
  acc2f = _msg(hh2, ee2, src, dst).reshape(NC, H, NPAD, C)
  acc2a = acc2f[0, :, :N].transpose(1, 0, 2).reshape(N, D)
  acc2b = acc2f[1, :, :N].transpose(1, 0, 2).reshape(N, D)

  x3 = _dense3(acc2a, acc2b, asum2, asum2, SE2, h2, b2.reshape(1, D))
  ug, vg = _predict(x3, user_indices.astype(i32), item_indices.astype(i32))
  return _dot(ug, vg).reshape(BATCH)
